# baseline (speedup 1.0000x reference)
import functools

import jax
import jax.numpy as jnp
from jax import lax
from jax.experimental import pallas as pl
from jax.experimental.pallas import tpu as pltpu
from jax.experimental.pallas import tpu_sc as plsc

N_VALUES = 10
BATCH = 16384
DIM = 17
DIM_PAD = 32
NC = 2
NS = 16
NW = NC * NS
B_PER_W = BATCH // NW
N_IDX_CHUNKS = B_PER_W // 128

_mesh = plsc.VectorSubcoreMesh(core_axis_name="c", subcore_axis_name="s")


def _sc_lookup_body(xt_hbm, w_hbm, out_hbm, x_v, idx_v, rows_v, out_v, sem):
    wid = lax.axis_index("s") * NC + lax.axis_index("c")
    base = wid * B_PER_W
    pltpu.sync_copy(xt_hbm.at[:, pl.ds(base, B_PER_W)], x_v)

    for i in range(B_PER_W // 16):
        s = pl.ds(i * 16, 16)
        acc = x_v[0, s]
        for j in range(1, 5):
            acc = acc * N_VALUES + x_v[j, s]
        idx_v[i // 8, pl.ds((i % 8) * 16, 16)] = acc

    copies = [
        pltpu.async_copy(
            w_hbm.at[idx_v.at[jnp.int32(j)]],
            rows_v.at[pl.ds(j * 128, 128)],
            sem,
        )
        for j in range(N_IDX_CHUNKS)
    ]
    for c in copies:
        c.wait()

    def conv_row(r, carry):
        out_v[r, pl.ds(0, 16)] = rows_v[r, pl.ds(0, 16)].astype(jnp.int32) + 1
        out_v[r, pl.ds(1, 16)] = rows_v[r, pl.ds(1, 16)].astype(jnp.int32) + 1
        return carry

    lax.fori_loop(0, B_PER_W, conv_row, 0)

    pltpu.sync_copy(out_v, out_hbm.at[pl.ds(base, B_PER_W)])


_sc_lookup = functools.partial(
    pl.kernel,
    mesh=_mesh,
    out_type=jax.ShapeDtypeStruct((BATCH, DIM), jnp.int32),
    scratch_types=[
        pltpu.VMEM((5, B_PER_W), jnp.int32),
        pltpu.VMEM((N_IDX_CHUNKS, 128), jnp.int32),
        pltpu.VMEM((B_PER_W, DIM_PAD), jnp.float32),
        pltpu.VMEM((B_PER_W, DIM), jnp.int32),
        pltpu.SemaphoreType.DMA,
    ],
    compiler_params=pltpu.CompilerParams(use_tc_tiling_on_sc=False),
)(_sc_lookup_body)


def kernel(x, W):
    xt32 = x.astype(jnp.int32).T
    w_pad = jnp.pad(W, ((0, 0), (0, DIM_PAD - DIM)))
    g32 = _sc_lookup(xt32, w_pad)
    g = g32.astype(jnp.int64)
    zeros = jnp.zeros((x.shape[0], W.shape[1]), dtype=jnp.float32)
    return (g, zeros, zeros)


# speedup vs baseline: 1.0933x; 1.0933x over previous
import functools

import jax
import jax.numpy as jnp
import numpy as np
from jax import lax
from jax.experimental import pallas as pl
from jax.experimental.pallas import tpu as pltpu
from jax.experimental.pallas import tpu_sc as plsc
from jax._src import config as _jax_config

N_VALUES = 10
BATCH = 16384
DIM = 17
V_ROWS = 100000
W2_ROWS = V_ROWS * DIM // 16
NC = 2
NS = 16
NW = NC * NS
B_PER_W = BATCH // NW
N_CHUNKS = B_PER_W // 128
N_GROUPS = B_PER_W // 16

_mesh = plsc.VectorSubcoreMesh(core_axis_name="c", subcore_axis_name="s")


def _sc_lookup_body(xt_hbm, w2_hbm, out_hbm, x_v, idx_v, o_v, ab_v, out_v, sem):
    wid = lax.axis_index("s") * NC + lax.axis_index("c")
    base = wid * B_PER_W
    pltpu.sync_copy(xt_hbm.at[:, pl.ds(base, B_PER_W)], x_v)

    lanes = lax.iota(jnp.int32, 16)
    for i in range(N_GROUPS):
        s16 = pl.ds(i * 16, 16)
        acc = x_v[0, s16]
        for j in range(1, 5):
            acc = acc * N_VALUES + x_v[j, s16]
        t = acc * DIM
        a = t >> 4
        o_v[s16] = t & 15
        idx_v[i // 8, pl.ds((i % 8) * 16, 16)] = a
        idx_v[4 + i // 8, pl.ds((i % 8) * 16, 16)] = a + 1

    copies = [
        pltpu.async_copy(
            w2_hbm.at[idx_v.at[jnp.int32(j)]],
            ab_v.at[pl.ds(j * 128, 128)],
            sem,
        )
        for j in range(2 * N_CHUNKS)
    ]
    for c in copies:
        c.wait()

    @pl.loop(np.int32(0), np.int32(B_PER_W), step=np.int32(16))
    def extract_group(b):
        r16 = b + lanes
        o16 = o_v[pl.ds(b, 16)]
        for c in range(DIM):
            s = o16 + c
            row16 = r16 + (s >> 4) * B_PER_W
            col16 = s & 15
            v = plsc.load_gather(ab_v, [row16, col16])
            cc = jnp.full((16,), c, jnp.int32)
            plsc.store_scatter(out_v, [r16, cc], v.astype(jnp.int32) + 1)

    pltpu.sync_copy(out_v, out_hbm.at[pl.ds(base, B_PER_W)])


_sc_lookup = functools.partial(
    pl.kernel,
    mesh=_mesh,
    out_type=jax.ShapeDtypeStruct((BATCH, DIM), jnp.int32),
    scratch_types=[
        pltpu.VMEM((5, B_PER_W), jnp.int32),
        pltpu.VMEM((2 * N_CHUNKS, 128), jnp.int32),
        pltpu.VMEM((B_PER_W,), jnp.int32),
        pltpu.VMEM((2 * B_PER_W, 16), jnp.float32),
        pltpu.VMEM((B_PER_W, DIM), jnp.int32),
        pltpu.SemaphoreType.DMA,
    ],
    compiler_params=pltpu.CompilerParams(
        use_tc_tiling_on_sc=False, needs_layout_passes=False
    ),
)(_sc_lookup_body)


def kernel(x, W):
    xw = x.astype(jnp.int32).T
    w2 = W.reshape(W2_ROWS, 16)
    with _jax_config.enable_x64(False):
        g32 = _sc_lookup(xw, w2)
    g = g32.astype(jnp.int64)
    zeros = jnp.zeros((x.shape[0], W.shape[1]), dtype=jnp.float32)
    return (g, zeros, zeros)


# speedup vs baseline: 2.8773x; 2.6318x over previous
import functools

import jax
import jax.numpy as jnp
import numpy as np
from jax import lax
from jax.experimental import pallas as pl
from jax.experimental.pallas import tpu as pltpu
from jax.experimental.pallas import tpu_sc as plsc
from jax._src import config as _jax_config

N_VALUES = 10
BATCH = 16384
DIM = 17
V_ROWS = 100000
W2_ROWS = V_ROWS * DIM // 16
NC = 2
NS = 16
NW = NC * NS
B_PER_W = BATCH // NW
N_CHUNKS = B_PER_W // 128
N_GROUPS = B_PER_W // 16

_mesh = plsc.VectorSubcoreMesh(core_axis_name="c", subcore_axis_name="s")


def _sc_lookup_body(xt_hbm, w2_hbm, out_hbm, x_v, idx_v, o_v, ab_v, out_v, sem):
    wid = lax.axis_index("s") * NC + lax.axis_index("c")
    base = wid * B_PER_W
    pltpu.sync_copy(xt_hbm.at[:, pl.ds(base, B_PER_W)], x_v)

    lanes = lax.iota(jnp.int32, 16)
    for i in range(N_GROUPS):
        s16 = pl.ds(i * 16, 16)
        acc = x_v[0, s16]
        for j in range(1, 5):
            acc = acc * N_VALUES + x_v[j, s16]
        t = acc * DIM
        a = t >> 4
        o_v[s16] = t & 15
        idx_v[i // 8, pl.ds((i % 8) * 16, 16)] = a
        idx_v[4 + i // 8, pl.ds((i % 8) * 16, 16)] = a + 1

    copies = [
        pltpu.async_copy(
            w2_hbm.at[idx_v.at[jnp.int32(j)]],
            ab_v.at[pl.ds(j * 128, 128)],
            sem,
        )
        for j in range(2 * N_CHUNKS)
    ]
    for c in copies:
        c.wait()

    @pl.loop(np.int32(0), np.int32(B_PER_W), step=np.int32(16))
    def extract_group(b):
        r16 = b + lanes
        o16 = o_v[pl.ds(b, 16)]
        for c in range(DIM):
            s = o16 + c
            row16 = r16 + (s >> 4) * B_PER_W
            col16 = s & 15
            v = plsc.load_gather(ab_v, [row16, col16])
            out_v[c, pl.ds(b, 16)] = v.astype(jnp.int32) + 1

    pltpu.sync_copy(out_v, out_hbm.at[:, pl.ds(base, B_PER_W)])


_sc_lookup = functools.partial(
    pl.kernel,
    mesh=_mesh,
    out_type=jax.ShapeDtypeStruct((DIM, BATCH), jnp.int32),
    scratch_types=[
        pltpu.VMEM((5, B_PER_W), jnp.int32),
        pltpu.VMEM((2 * N_CHUNKS, 128), jnp.int32),
        pltpu.VMEM((B_PER_W,), jnp.int32),
        pltpu.VMEM((2 * B_PER_W, 16), jnp.float32),
        pltpu.VMEM((DIM, B_PER_W), jnp.int32),
        pltpu.SemaphoreType.DMA,
    ],
    compiler_params=pltpu.CompilerParams(
        use_tc_tiling_on_sc=False, needs_layout_passes=False
    ),
)(_sc_lookup_body)


def kernel(x, W):
    xw = x.astype(jnp.int32).T
    w2 = W.reshape(W2_ROWS, 16)
    with _jax_config.enable_x64(False):
        g32t = _sc_lookup(xw, w2)
    g = g32t.T.astype(jnp.int64)
    zeros = jnp.zeros((x.shape[0], W.shape[1]), dtype=jnp.float32)
    return (g, zeros, zeros)


# speedup vs baseline: 5.1528x; 1.7909x over previous
import functools

import jax
import jax.numpy as jnp
import numpy as np
from jax import lax
from jax.experimental import pallas as pl
from jax.experimental.pallas import tpu as pltpu
from jax.experimental.pallas import tpu_sc as plsc
from jax._src import config as _jax_config

N_VALUES = 10
BATCH = 16384
DIM = 17
V_ROWS = 100000
WT_ROWS = V_ROWS * DIM // 8
ROWS_PER_COL = V_ROWS // 8
NC = 2
NS = 16
NW = NC * NS
B_PER_W = BATCH // NW
N_GROUPS = B_PER_W // 16
N_CHUNKS = DIM * B_PER_W // 128

_mesh = plsc.VectorSubcoreMesh(core_axis_name="c", subcore_axis_name="s")


def _sc_lookup_body(xt_hbm, wt_hbm, out_hbm, x_v, idx_v, o_v, buf_v, out_v, sem):
    wid = lax.axis_index("s") * NC + lax.axis_index("c")
    base = wid * B_PER_W
    pltpu.sync_copy(xt_hbm.at[:, pl.ds(base, B_PER_W)], x_v)

    lanes = lax.iota(jnp.int32, 16)
    for i in range(N_GROUPS):
        s16 = pl.ds(i * 16, 16)
        acc = x_v[0, s16]
        for j in range(1, 5):
            acc = acc * N_VALUES + x_v[j, s16]
        o_v[s16] = acc & 7
        hi = acc >> 3
        for c in range(DIM):
            idx_v[4 * c + i // 8, pl.ds((i % 8) * 16, 16)] = (
                hi + c * ROWS_PER_COL
            )

    copies = [
        pltpu.async_copy(
            wt_hbm.at[idx_v.at[jnp.int32(j)]],
            buf_v.at[pl.ds(j * 128, 128)],
            sem,
        )
        for j in range(N_CHUNKS)
    ]
    for c in copies:
        c.wait()

    @pl.loop(np.int32(0), np.int32(B_PER_W), step=np.int32(16))
    def extract_group(b):
        r16 = b + lanes
        o16 = o_v[pl.ds(b, 16)]
        for c in range(DIM):
            row16 = r16 + c * B_PER_W
            v = plsc.load_gather(buf_v, [row16, o16])
            out_v[c, pl.ds(b, 16)] = v.astype(jnp.int32) + 1

    pltpu.sync_copy(out_v, out_hbm.at[:, pl.ds(base, B_PER_W)])


_sc_lookup = functools.partial(
    pl.kernel,
    mesh=_mesh,
    out_type=jax.ShapeDtypeStruct((DIM, BATCH), jnp.int32),
    scratch_types=[
        pltpu.VMEM((5, B_PER_W), jnp.int32),
        pltpu.VMEM((N_CHUNKS, 128), jnp.int32),
        pltpu.VMEM((B_PER_W,), jnp.int32),
        pltpu.VMEM((DIM * B_PER_W, 8), jnp.float32),
        pltpu.VMEM((DIM, B_PER_W), jnp.int32),
        pltpu.SemaphoreType.DMA,
    ],
    compiler_params=pltpu.CompilerParams(
        use_tc_tiling_on_sc=False, needs_layout_passes=False
    ),
)(_sc_lookup_body)


def kernel(x, W):
    xw = x.astype(jnp.int32).T
    wt = W.T.reshape(WT_ROWS, 8)
    with _jax_config.enable_x64(False):
        g32t = _sc_lookup(xw, wt)
    g = g32t.T.astype(jnp.int64)
    zeros = jnp.zeros((x.shape[0], W.shape[1]), dtype=jnp.float32)
    return (g, zeros, zeros)
